# baseline (speedup 1.0000x reference)
import jax
import jax.numpy as jnp
from jax import lax
from jax.experimental import pallas as pl
from jax.experimental.pallas import tpu as pltpu
from jax.experimental.pallas import tpu_sc as plsc

NC, NS, L = 2, 16, 16
NW = NC * NS
B = 16384
D = 64
N = 1000000
BPW = B // NW
CH = 128

CE = 512
NFULL = N // CE
TAILW = N - NFULL * CE
NCHK = NFULL + 1
WIN = 1568
OFFPAD = 1984
NSLOT = 3 * B
DUMMY = NSLOT
SCR = NSLOT + 64
CAP = 64


def _sweep_body(ent_hbm, rel_hbm, sie_hbm, sse_hbm, offe_hbm,
                sir_hbm, ssr_hbm, offr_hbm, scr_hbm,
                off_v, widx_v, wslt_v, slab, tail_v, stage_v, sl_v,
                sem_l, sem_s):
    wid = lax.axis_index("s") * NC + lax.axis_index("c")
    lane = lax.iota(jnp.int32, L)

    def sload(ref, i):
        return ref[pl.ds(i, L)][0]

    acoef = [(lane + c * L) >> 3 for c in range(4)]
    rcoef = [(lane + c * L) & 7 for c in range(4)]
    c0 = (wid * NCHK) // NW
    c1 = ((wid + 1) * NCHK) // NW
    c1m = jnp.minimum(c1, NFULL)

    def fire_load(tbl, cc):
        for a in range(8):
            pltpu.async_copy(
                tbl.at[pl.ds(a * 8, 8), pl.ds(cc * CE, CE)],
                slab.at[cc & 1, a], sem_l)

    def wait_load(tbl, cc):
        for a in range(8):
            pltpu.make_async_copy(
                tbl.at[pl.ds(a * 8, 8), pl.ds(cc * CE, CE)],
                slab.at[cc & 1, a], sem_l).wait()

    def init_slots(p):
        for q in range(CAP // L):
            sl_v[p, pl.ds(q * L, L)] = jnp.full((L,), DUMMY, jnp.int32)

    def make_serve(from_tail):
        def serve(k, args):
            ws, n0, ebase, p = args
            kl = k - ws
            i = sload(widx_v, kl)
            slot = sload(wslt_v, kl)
            e = i - ebase
            m = (k - n0) & (CAP - 1)
            ev = jnp.full((L,), e, jnp.int32)
            for c in range(4):
                if from_tail:
                    vals = plsc.load_gather(tail_v, [acoef[c], rcoef[c], ev])
                else:
                    pv = jnp.full((L,), p, jnp.int32)
                    vals = plsc.load_gather(slab, [pv, acoef[c], rcoef[c], ev])
                stage_v[p, m, pl.ds(c * L, L)] = vals
            base16 = (m >> 4) * L
            cur = sl_v[p, pl.ds(base16, L)]
            sl_v[p, pl.ds(base16, L)] = jnp.where(
                lane == (m & (L - 1)), slot, cur)
            return args

        return serve

    serve_main = make_serve(False)
    serve_tail = make_serve(True)

    def sweep(tbl, si_hbm, ss_hbm, o_hbm):
        pltpu.sync_copy(o_hbm.at[pl.ds(0, OFFPAD)], off_v)
        ws = pl.multiple_of(sload(off_v, c0) & ~7, 8)
        pltpu.sync_copy(si_hbm.at[pl.ds(ws, WIN)], widx_v)
        pltpu.sync_copy(ss_hbm.at[pl.ds(ws, WIN)], wslt_v)
        for p in range(2):
            init_slots(p)
            pltpu.async_copy(stage_v.at[p], scr_hbm.at[sl_v.at[p]], sem_s)
        fire_load(tbl, c0)

        def chunk(cc, carry):
            p = cc & 1
            wait_load(tbl, cc)

            @pl.when(cc + 1 < c1m)
            def _():
                fire_load(tbl, cc + 1)

            pltpu.make_async_copy(
                stage_v.at[p], scr_hbm.at[sl_v.at[p]], sem_s).wait()
            init_slots(p)
            n0 = sload(off_v, cc)
            n1 = sload(off_v, cc + 1)
            lax.fori_loop(n0, n1, serve_main, (ws, n0, cc * CE, p))
            pltpu.async_copy(stage_v.at[p], scr_hbm.at[sl_v.at[p]], sem_s)
            return carry

        lax.fori_loop(c0, c1m, chunk, 0)
        for p in range(2):
            pltpu.make_async_copy(
                stage_v.at[p], scr_hbm.at[sl_v.at[p]], sem_s).wait()

        @pl.when(c1 == NCHK)
        def _():
            for a in range(8):
                pltpu.async_copy(
                    tbl.at[pl.ds(a * 8, 8), pl.ds(NFULL * CE, TAILW)],
                    tail_v.at[a], sem_l)
            for a in range(8):
                pltpu.make_async_copy(
                    tbl.at[pl.ds(a * 8, 8), pl.ds(NFULL * CE, TAILW)],
                    tail_v.at[a], sem_l).wait()
            init_slots(0)
            n0 = sload(off_v, NFULL)
            n1 = sload(off_v, NFULL + 1)
            lax.fori_loop(n0, n1, serve_tail,
                          (ws, n0, NFULL * CE, 0))
            pltpu.sync_copy(stage_v.at[0], scr_hbm.at[sl_v.at[0]])

    sweep(ent_hbm, sie_hbm, sse_hbm, offe_hbm)
    sweep(rel_hbm, sir_hbm, ssr_hbm, offr_hbm)


def _score_body(scr_hbm, out_hbm, hbuf, rbuf, tbuf, score_v,
                sem_h, sem_r, sem_t):
    wid = lax.axis_index("s") * NC + lax.axis_index("c")
    lane = lax.iota(jnp.int32, L)
    for j in range(BPW // CH):
        s0 = wid * BPW + j * CH
        cph = pltpu.async_copy(scr_hbm.at[pl.ds(s0, CH)], hbuf, sem_h)
        cpt = pltpu.async_copy(scr_hbm.at[pl.ds(B + s0, CH)], tbuf, sem_t)
        cpr = pltpu.async_copy(scr_hbm.at[pl.ds(2 * B + s0, CH)], rbuf, sem_r)
        cph.wait()
        cpt.wait()
        cpr.wait()

        def compute(g, carry):
            rows = g * L + lane
            acc = jnp.zeros((L,), jnp.float32)
            for d in range(D):
                dv = jnp.full((L,), d, jnp.int32)
                hv = plsc.load_gather(hbuf, [rows, dv])
                rv = plsc.load_gather(rbuf, [rows, dv])
                tv = plsc.load_gather(tbuf, [rows, dv])
                acc = acc + jnp.abs(hv + rv - tv)
            score_v[pl.ds(g * L, L)] = acc
            return carry

        lax.fori_loop(0, CH // L, compute, 0)
        pltpu.sync_copy(score_v, out_hbm.at[pl.ds(s0, CH)])


def kernel(sample, entity_embedding, relation_embedding):
    ent_t = entity_embedding.T
    rel_t = relation_embedding.T
    idx_e = jnp.concatenate([sample[:, 0], sample[:, 2]])
    perm_e = jnp.argsort(idx_e)
    sie = jnp.concatenate([idx_e[perm_e],
                           jnp.full((WIN,), jnp.int32(2**30))])
    sse = jnp.concatenate([perm_e.astype(jnp.int32),
                           jnp.zeros((WIN,), jnp.int32)])
    idx_r = sample[:, 1]
    perm_r = jnp.argsort(idx_r)
    sir = jnp.concatenate([idx_r[perm_r],
                           jnp.full((WIN,), jnp.int32(2**30))])
    ssr = jnp.concatenate([(perm_r + 2 * B).astype(jnp.int32),
                           jnp.zeros((WIN,), jnp.int32)])
    bounds = jnp.minimum(jnp.arange(NCHK + 1, dtype=jnp.int32) * CE, N)
    offe = jnp.concatenate([
        jnp.searchsorted(sie[:2 * B], bounds).astype(jnp.int32),
        jnp.full((OFFPAD - NCHK - 1,), jnp.int32(2 * B))])
    offr = jnp.concatenate([
        jnp.searchsorted(sir[:B], bounds).astype(jnp.int32),
        jnp.full((OFFPAD - NCHK - 1,), jnp.int32(B))])

    mesh = plsc.VectorSubcoreMesh(
        core_axis_name="c", subcore_axis_name="s",
        num_cores=NC, num_subcores=NS)
    params = pltpu.CompilerParams(
        needs_layout_passes=False, use_tc_tiling_on_sc=True)

    gathered = pl.kernel(
        _sweep_body,
        out_type=jax.ShapeDtypeStruct((SCR, 2 * D), jnp.float32),
        mesh=mesh,
        compiler_params=params,
        scratch_types=[
            pltpu.VMEM((OFFPAD,), jnp.int32),
            pltpu.VMEM((WIN,), jnp.int32),
            pltpu.VMEM((WIN,), jnp.int32),
            pltpu.VMEM((2, 8, 8, CE), jnp.float32),
            pltpu.VMEM((8, 8, TAILW), jnp.float32),
            pltpu.VMEM((2, CAP, 2 * D), jnp.float32),
            pltpu.VMEM((2, CAP), jnp.int32),
            pltpu.SemaphoreType.DMA,
            pltpu.SemaphoreType.DMA,
        ],
    )(ent_t, rel_t, sie, sse, offe, sir, ssr, offr)

    score = pl.kernel(
        _score_body,
        out_type=jax.ShapeDtypeStruct((B,), jnp.float32),
        mesh=mesh,
        compiler_params=params,
        scratch_types=[
            pltpu.VMEM((CH, 2 * D), jnp.float32),
            pltpu.VMEM((CH, 2 * D), jnp.float32),
            pltpu.VMEM((CH, 2 * D), jnp.float32),
            pltpu.VMEM((CH,), jnp.float32),
            pltpu.SemaphoreType.DMA,
            pltpu.SemaphoreType.DMA,
            pltpu.SemaphoreType.DMA,
        ],
    )(gathered)
    return score.reshape(B, 1)


# speedup vs baseline: 11.0643x; 11.0643x over previous
import jax
import jax.numpy as jnp
from jax import lax
from jax.experimental import pallas as pl
from jax.experimental.pallas import tpu as pltpu
from jax.experimental.pallas import tpu_sc as plsc

NC, NS, L = 2, 16, 16
NW = NC * NS
B = 16384
D = 64
N = 1000000
BPW = B // NW
CH = 128

CE = 512
NFULL = N // CE
TAILW = N - NFULL * CE
NCHK = NFULL + 1
WIN = 1568
OFFPAD = 1984
NSLOT = 3 * B
CAP = 64
SCR = NSLOT + NW * CAP


def _sweep_body(ent_hbm, rel_hbm, sie_hbm, sse_hbm, offe_hbm,
                sir_hbm, ssr_hbm, offr_hbm, scr_hbm,
                off_v, widx_v, wslt_v, slab, tail_v, stage_v, sl_v,
                sem_l, sem_s):
    wid = lax.axis_index("s") * NC + lax.axis_index("c")
    lane = lax.iota(jnp.int32, L)

    def sload(ref, i):
        return ref[pl.ds(i, L)][0]

    acoef = [(lane + c * L) >> 3 for c in range(4)]
    rcoef = [(lane + c * L) & 7 for c in range(4)]
    c0 = (wid * NCHK) // NW
    c1 = ((wid + 1) * NCHK) // NW
    c1m = jnp.minimum(c1, NFULL)

    def fire_load(tbl, cc):
        for a in range(8):
            pltpu.async_copy(
                tbl.at[pl.ds(a * 8, 8), pl.ds(cc * CE, CE)],
                slab.at[cc & 1, a], sem_l)

    def wait_load(tbl, cc):
        for a in range(8):
            pltpu.make_async_copy(
                tbl.at[pl.ds(a * 8, 8), pl.ds(cc * CE, CE)],
                slab.at[cc & 1, a], sem_l).wait()

    dummy0 = NSLOT + wid * CAP

    def init_slots(p):
        for q in range(CAP // L):
            sl_v[p, pl.ds(q * L, L)] = dummy0 + q * L + lane

    def make_serve(from_tail):
        def serve(k, args):
            ws, n0, ebase, p = args
            kl = k - ws
            i = sload(widx_v, kl)
            slot = sload(wslt_v, kl)
            e = i - ebase
            m = (k - n0) & (CAP - 1)
            ev = jnp.full((L,), e, jnp.int32)
            for c in range(4):
                if from_tail:
                    vals = plsc.load_gather(tail_v, [acoef[c], rcoef[c], ev])
                else:
                    pv = jnp.full((L,), p, jnp.int32)
                    vals = plsc.load_gather(slab, [pv, acoef[c], rcoef[c], ev])
                stage_v[p, m, pl.ds(c * L, L)] = vals
            base16 = (m >> 4) * L
            cur = sl_v[p, pl.ds(base16, L)]
            sl_v[p, pl.ds(base16, L)] = jnp.where(
                lane == (m & (L - 1)), slot, cur)
            return args

        return serve

    serve_main = make_serve(False)
    serve_tail = make_serve(True)

    def sweep(tbl, si_hbm, ss_hbm, o_hbm):
        pltpu.sync_copy(o_hbm.at[pl.ds(0, OFFPAD)], off_v)
        ws = pl.multiple_of(sload(off_v, c0) & ~7, 8)
        pltpu.sync_copy(si_hbm.at[pl.ds(ws, WIN)], widx_v)
        pltpu.sync_copy(ss_hbm.at[pl.ds(ws, WIN)], wslt_v)
        for p in range(2):
            init_slots(p)
            pltpu.async_copy(stage_v.at[p], scr_hbm.at[sl_v.at[p]], sem_s)
        fire_load(tbl, c0)

        def chunk(cc, carry):
            p = cc & 1
            wait_load(tbl, cc)

            @pl.when(cc + 1 < c1m)
            def _():
                fire_load(tbl, cc + 1)

            pltpu.make_async_copy(
                stage_v.at[p], scr_hbm.at[sl_v.at[p]], sem_s).wait()
            init_slots(p)
            n0 = sload(off_v, cc)
            n1 = sload(off_v, cc + 1)
            lax.fori_loop(n0, n1, serve_main, (ws, n0, cc * CE, p))
            pltpu.async_copy(stage_v.at[p], scr_hbm.at[sl_v.at[p]], sem_s)
            return carry

        lax.fori_loop(c0, c1m, chunk, 0)
        for p in range(2):
            pltpu.make_async_copy(
                stage_v.at[p], scr_hbm.at[sl_v.at[p]], sem_s).wait()

        @pl.when(c1 == NCHK)
        def _():
            for a in range(8):
                pltpu.async_copy(
                    tbl.at[pl.ds(a * 8, 8), pl.ds(NFULL * CE, TAILW)],
                    tail_v.at[a], sem_l)
            for a in range(8):
                pltpu.make_async_copy(
                    tbl.at[pl.ds(a * 8, 8), pl.ds(NFULL * CE, TAILW)],
                    tail_v.at[a], sem_l).wait()
            init_slots(0)
            n0 = sload(off_v, NFULL)
            n1 = sload(off_v, NFULL + 1)
            lax.fori_loop(n0, n1, serve_tail,
                          (ws, n0, NFULL * CE, 0))
            pltpu.sync_copy(stage_v.at[0], scr_hbm.at[sl_v.at[0]])

    sweep(ent_hbm, sie_hbm, sse_hbm, offe_hbm)
    sweep(rel_hbm, sir_hbm, ssr_hbm, offr_hbm)


def _score_body(scr_hbm, out_hbm, hbuf, rbuf, tbuf, score_v,
                sem_h, sem_r, sem_t):
    wid = lax.axis_index("s") * NC + lax.axis_index("c")
    lane = lax.iota(jnp.int32, L)
    for j in range(BPW // CH):
        s0 = wid * BPW + j * CH
        cph = pltpu.async_copy(scr_hbm.at[pl.ds(s0, CH)], hbuf, sem_h)
        cpt = pltpu.async_copy(scr_hbm.at[pl.ds(B + s0, CH)], tbuf, sem_t)
        cpr = pltpu.async_copy(scr_hbm.at[pl.ds(2 * B + s0, CH)], rbuf, sem_r)
        cph.wait()
        cpt.wait()
        cpr.wait()

        def compute(g, carry):
            rows = g * L + lane
            acc = jnp.zeros((L,), jnp.float32)
            for d in range(D):
                dv = jnp.full((L,), d, jnp.int32)
                hv = plsc.load_gather(hbuf, [rows, dv])
                rv = plsc.load_gather(rbuf, [rows, dv])
                tv = plsc.load_gather(tbuf, [rows, dv])
                acc = acc + jnp.abs(hv + rv - tv)
            score_v[pl.ds(g * L, L)] = acc
            return carry

        lax.fori_loop(0, CH // L, compute, 0)
        pltpu.sync_copy(score_v, out_hbm.at[pl.ds(s0, CH)])


def kernel(sample, entity_embedding, relation_embedding):
    ent_t = entity_embedding.T
    rel_t = relation_embedding.T
    idx_e = jnp.concatenate([sample[:, 0], sample[:, 2]])
    perm_e = jnp.argsort(idx_e)
    sie = jnp.concatenate([idx_e[perm_e],
                           jnp.full((WIN,), jnp.int32(2**30))])
    sse = jnp.concatenate([perm_e.astype(jnp.int32),
                           jnp.zeros((WIN,), jnp.int32)])
    idx_r = sample[:, 1]
    perm_r = jnp.argsort(idx_r)
    sir = jnp.concatenate([idx_r[perm_r],
                           jnp.full((WIN,), jnp.int32(2**30))])
    ssr = jnp.concatenate([(perm_r + 2 * B).astype(jnp.int32),
                           jnp.zeros((WIN,), jnp.int32)])
    bounds = jnp.minimum(jnp.arange(NCHK + 1, dtype=jnp.int32) * CE, N)
    offe = jnp.concatenate([
        jnp.searchsorted(sie[:2 * B], bounds).astype(jnp.int32),
        jnp.full((OFFPAD - NCHK - 1,), jnp.int32(2 * B))])
    offr = jnp.concatenate([
        jnp.searchsorted(sir[:B], bounds).astype(jnp.int32),
        jnp.full((OFFPAD - NCHK - 1,), jnp.int32(B))])

    mesh = plsc.VectorSubcoreMesh(
        core_axis_name="c", subcore_axis_name="s",
        num_cores=NC, num_subcores=NS)
    params = pltpu.CompilerParams(
        needs_layout_passes=False, use_tc_tiling_on_sc=True)

    gathered = pl.kernel(
        _sweep_body,
        out_type=jax.ShapeDtypeStruct((SCR, 2 * D), jnp.float32),
        mesh=mesh,
        compiler_params=params,
        scratch_types=[
            pltpu.VMEM((OFFPAD,), jnp.int32),
            pltpu.VMEM((WIN,), jnp.int32),
            pltpu.VMEM((WIN,), jnp.int32),
            pltpu.VMEM((2, 8, 8, CE), jnp.float32),
            pltpu.VMEM((8, 8, TAILW), jnp.float32),
            pltpu.VMEM((2, CAP, 2 * D), jnp.float32),
            pltpu.VMEM((2, CAP), jnp.int32),
            pltpu.SemaphoreType.DMA,
            pltpu.SemaphoreType.DMA,
        ],
    )(ent_t, rel_t, sie, sse, offe, sir, ssr, offr)

    score = pl.kernel(
        _score_body,
        out_type=jax.ShapeDtypeStruct((B,), jnp.float32),
        mesh=mesh,
        compiler_params=params,
        scratch_types=[
            pltpu.VMEM((CH, 2 * D), jnp.float32),
            pltpu.VMEM((CH, 2 * D), jnp.float32),
            pltpu.VMEM((CH, 2 * D), jnp.float32),
            pltpu.VMEM((CH,), jnp.float32),
            pltpu.SemaphoreType.DMA,
            pltpu.SemaphoreType.DMA,
            pltpu.SemaphoreType.DMA,
        ],
    )(gathered)
    return score.reshape(B, 1)


# speedup vs baseline: 25.6145x; 2.3151x over previous
import jax
import jax.numpy as jnp
from jax import lax
from jax.experimental import pallas as pl
from jax.experimental.pallas import tpu as pltpu
from jax.experimental.pallas import tpu_sc as plsc

NC, NS, L = 2, 16, 16
NW = NC * NS
B = 16384
D = 64
N = 1000000
BPW = B // NW
CH = 128

CE = 512
NFULL = N // CE
TAILW = N - NFULL * CE
NCHK = NFULL + 1
WIN = 1568
OFFPAD = 1984
NSLOT = 3 * B
CAP = 64
SCR = NSLOT + NW * CAP


def _sweep_body(ent_hbm, rel_hbm, sie_hbm, sse_hbm, offe_hbm,
                sir_hbm, ssr_hbm, offr_hbm, scr_hbm,
                off_v, widx_v, wslt_v, slab, tail_v, stage_v, sl_v,
                sem_l, sem_s):
    wid = lax.axis_index("s") * NC + lax.axis_index("c")
    lane = lax.iota(jnp.int32, L)

    def sload(ref, i):
        return ref[pl.ds(i, L)][0]

    acoef = [(lane + c * L) >> 3 for c in range(4)]
    rcoef = [(lane + c * L) & 7 for c in range(4)]
    c0 = (wid * NCHK) // NW
    c1 = ((wid + 1) * NCHK) // NW
    c1m = jnp.minimum(c1, NFULL)

    def fire_load(tbl, cc):
        for a in range(8):
            pltpu.async_copy(
                tbl.at[pl.ds(a * 8, 8), pl.ds(cc * CE, CE)],
                slab.at[cc & 1, a], sem_l)

    def wait_load(tbl, cc):
        for a in range(8):
            pltpu.make_async_copy(
                tbl.at[pl.ds(a * 8, 8), pl.ds(cc * CE, CE)],
                slab.at[cc & 1, a], sem_l).wait()

    dummy0 = NSLOT + wid * CAP

    def init_slots(p):
        for q in range(CAP // L):
            sl_v[p, pl.ds(q * L, L)] = dummy0 + q * L + lane

    def make_serve(from_tail):
        def serve(k, args):
            ws, n0, ebase, p = args
            kl = k - ws
            i = sload(widx_v, kl)
            slot = sload(wslt_v, kl)
            e = i - ebase
            m = (k - n0) & (CAP - 1)
            ev = jnp.full((L,), e, jnp.int32)
            for c in range(4):
                if from_tail:
                    vals = plsc.load_gather(tail_v, [acoef[c], rcoef[c], ev])
                else:
                    pv = jnp.full((L,), p, jnp.int32)
                    vals = plsc.load_gather(slab, [pv, acoef[c], rcoef[c], ev])
                stage_v[p, m, pl.ds(c * L, L)] = vals
            base16 = (m >> 4) * L
            cur = sl_v[p, pl.ds(base16, L)]
            sl_v[p, pl.ds(base16, L)] = jnp.where(
                lane == (m & (L - 1)), slot, cur)
            return args

        return serve

    serve_main = make_serve(False)
    serve_tail = make_serve(True)

    def sweep(tbl, si_hbm, ss_hbm, o_hbm):
        pltpu.sync_copy(o_hbm.at[pl.ds(0, OFFPAD)], off_v)
        ws = pl.multiple_of(sload(off_v, c0) & ~7, 8)
        pltpu.sync_copy(si_hbm.at[pl.ds(ws, WIN)], widx_v)
        pltpu.sync_copy(ss_hbm.at[pl.ds(ws, WIN)], wslt_v)
        for p in range(2):
            init_slots(p)
            pltpu.async_copy(stage_v.at[p], scr_hbm.at[sl_v.at[p]], sem_s)
        fire_load(tbl, c0)

        def chunk(cc, carry):
            p = cc & 1
            wait_load(tbl, cc)

            @pl.when(cc + 1 < c1m)
            def _():
                fire_load(tbl, cc + 1)

            pltpu.make_async_copy(
                stage_v.at[p], scr_hbm.at[sl_v.at[p]], sem_s).wait()
            init_slots(p)
            n0 = sload(off_v, cc)
            n1 = sload(off_v, cc + 1)
            lax.fori_loop(n0, n1, serve_main, (ws, n0, cc * CE, p))
            pltpu.async_copy(stage_v.at[p], scr_hbm.at[sl_v.at[p]], sem_s)
            return carry

        lax.fori_loop(c0, c1m, chunk, 0)
        for p in range(2):
            pltpu.make_async_copy(
                stage_v.at[p], scr_hbm.at[sl_v.at[p]], sem_s).wait()

        @pl.when(c1 == NCHK)
        def _():
            for a in range(8):
                pltpu.async_copy(
                    tbl.at[pl.ds(a * 8, 8), pl.ds(NFULL * CE, TAILW)],
                    tail_v.at[a], sem_l)
            for a in range(8):
                pltpu.make_async_copy(
                    tbl.at[pl.ds(a * 8, 8), pl.ds(NFULL * CE, TAILW)],
                    tail_v.at[a], sem_l).wait()
            init_slots(0)
            n0 = sload(off_v, NFULL)
            n1 = sload(off_v, NFULL + 1)
            lax.fori_loop(n0, n1, serve_tail,
                          (ws, n0, NFULL * CE, 0))
            pltpu.sync_copy(stage_v.at[0], scr_hbm.at[sl_v.at[0]])

    sweep(ent_hbm, sie_hbm, sse_hbm, offe_hbm)
    sweep(rel_hbm, sir_hbm, ssr_hbm, offr_hbm)


def _score_body(scr_hbm, out_hbm, hbuf, rbuf, tbuf, score_v, tpose_v,
                sem_h, sem_r, sem_t):
    wid = lax.axis_index("s") * NC + lax.axis_index("c")
    lane = lax.iota(jnp.int32, L)
    for j in range(BPW // CH):
        s0 = wid * BPW + j * CH
        cph = pltpu.async_copy(scr_hbm.at[pl.ds(s0, CH)], hbuf, sem_h)
        cpt = pltpu.async_copy(scr_hbm.at[pl.ds(B + s0, CH)], tbuf, sem_t)
        cpr = pltpu.async_copy(scr_hbm.at[pl.ds(2 * B + s0, CH)], rbuf, sem_r)
        cph.wait()
        cpt.wait()
        cpr.wait()

        def compute(g, carry):
            for ii in range(L):
                i = g * L + ii
                acc = None
                for c in range(D // L):
                    h = hbuf[i, pl.ds(c * L, L)]
                    r = rbuf[i, pl.ds(c * L, L)]
                    t = tbuf[i, pl.ds(c * L, L)]
                    a = jnp.abs(h + r - t)
                    acc = a if acc is None else acc + a
                tpose_v[ii, pl.ds(0, L)] = acc
            svec = None
            for c in range(L):
                part = plsc.load_gather(
                    tpose_v, [lane, jnp.full((L,), c, jnp.int32)])
                svec = part if svec is None else svec + part
            score_v[pl.ds(g * L, L)] = svec
            return carry

        lax.fori_loop(0, CH // L, compute, 0)
        pltpu.sync_copy(score_v, out_hbm.at[pl.ds(s0, CH)])


def kernel(sample, entity_embedding, relation_embedding):
    ent_t = entity_embedding.T
    rel_t = relation_embedding.T
    idx_e = jnp.concatenate([sample[:, 0], sample[:, 2]])
    perm_e = jnp.argsort(idx_e)
    sie = jnp.concatenate([idx_e[perm_e],
                           jnp.full((WIN,), jnp.int32(2**30))])
    sse = jnp.concatenate([perm_e.astype(jnp.int32),
                           jnp.zeros((WIN,), jnp.int32)])
    idx_r = sample[:, 1]
    perm_r = jnp.argsort(idx_r)
    sir = jnp.concatenate([idx_r[perm_r],
                           jnp.full((WIN,), jnp.int32(2**30))])
    ssr = jnp.concatenate([(perm_r + 2 * B).astype(jnp.int32),
                           jnp.zeros((WIN,), jnp.int32)])
    cnt_e = jnp.zeros((NCHK,), jnp.int32).at[idx_e >> 9].add(1)
    cnt_r = jnp.zeros((NCHK,), jnp.int32).at[idx_r >> 9].add(1)
    offe = jnp.concatenate([
        jnp.zeros((1,), jnp.int32), jnp.cumsum(cnt_e, dtype=jnp.int32),
        jnp.full((OFFPAD - NCHK - 1,), jnp.int32(2 * B))])
    offr = jnp.concatenate([
        jnp.zeros((1,), jnp.int32), jnp.cumsum(cnt_r, dtype=jnp.int32),
        jnp.full((OFFPAD - NCHK - 1,), jnp.int32(B))])

    mesh = plsc.VectorSubcoreMesh(
        core_axis_name="c", subcore_axis_name="s",
        num_cores=NC, num_subcores=NS)
    params = pltpu.CompilerParams(
        needs_layout_passes=False, use_tc_tiling_on_sc=True)

    gathered = pl.kernel(
        _sweep_body,
        out_type=jax.ShapeDtypeStruct((SCR, 2 * D), jnp.float32),
        mesh=mesh,
        compiler_params=params,
        scratch_types=[
            pltpu.VMEM((OFFPAD,), jnp.int32),
            pltpu.VMEM((WIN,), jnp.int32),
            pltpu.VMEM((WIN,), jnp.int32),
            pltpu.VMEM((2, 8, 8, CE), jnp.float32),
            pltpu.VMEM((8, 8, TAILW), jnp.float32),
            pltpu.VMEM((2, CAP, 2 * D), jnp.float32),
            pltpu.VMEM((2, CAP), jnp.int32),
            pltpu.SemaphoreType.DMA,
            pltpu.SemaphoreType.DMA,
        ],
    )(ent_t, rel_t, sie, sse, offe, sir, ssr, offr)

    score = pl.kernel(
        _score_body,
        out_type=jax.ShapeDtypeStruct((B,), jnp.float32),
        mesh=mesh,
        compiler_params=pltpu.CompilerParams(
            needs_layout_passes=False, use_tc_tiling_on_sc=False),
        scratch_types=[
            pltpu.VMEM((CH, 2 * D), jnp.float32),
            pltpu.VMEM((CH, 2 * D), jnp.float32),
            pltpu.VMEM((CH, 2 * D), jnp.float32),
            pltpu.VMEM((CH,), jnp.float32),
            pltpu.VMEM((L, L + 1), jnp.float32),
            pltpu.SemaphoreType.DMA,
            pltpu.SemaphoreType.DMA,
            pltpu.SemaphoreType.DMA,
        ],
    )(gathered)
    return score.reshape(B, 1)


# speedup vs baseline: 28.2316x; 1.1022x over previous
import jax
import jax.numpy as jnp
from jax import lax
from jax.experimental import pallas as pl
from jax.experimental.pallas import tpu as pltpu
from jax.experimental.pallas import tpu_sc as plsc

NC, NS, L = 2, 16, 16
NW = NC * NS
B = 16384
D = 64
N = 1000000
BPW = B // NW
CH = 128

CE = 768
NFULL = N // CE
TAILW = N - NFULL * CE
NCHK = NFULL + 1
WIN = 1568
OFFPAD = 1984
NSLOT = 3 * B
CAP = 64
SCR = NSLOT + NW * CAP


def _sweep_body(ent_hbm, rel_hbm, sie_hbm, sse_hbm, offe_hbm,
                sir_hbm, ssr_hbm, offr_hbm, scr_hbm,
                off_v, widx_v, wslt_v, slab, tail_v, stage_v, sl_v,
                sem_l, sem_s):
    wid = lax.axis_index("s") * NC + lax.axis_index("c")
    lane = lax.iota(jnp.int32, L)

    def sload(ref, i):
        return ref[pl.ds(i, L)][0]

    acoef = [(lane + c * L) >> 3 for c in range(4)]
    rcoef = [(lane + c * L) & 7 for c in range(4)]
    c0 = (wid * NCHK) // NW
    c1 = ((wid + 1) * NCHK) // NW
    c1m = jnp.minimum(c1, NFULL)

    def fire_load(tbl, cc):
        for a in range(8):
            pltpu.async_copy(
                tbl.at[pl.ds(a * 8, 8), pl.ds(cc * CE, CE)],
                slab.at[cc & 1, a], sem_l)

    def wait_load(tbl, cc):
        for a in range(8):
            pltpu.make_async_copy(
                tbl.at[pl.ds(a * 8, 8), pl.ds(cc * CE, CE)],
                slab.at[cc & 1, a], sem_l).wait()

    dummy0 = NSLOT + wid * CAP

    def init_slots(p):
        for q in range(CAP // L):
            sl_v[p, pl.ds(q * L, L)] = dummy0 + q * L + lane

    def make_serve(from_tail):
        def serve(k, args):
            ws, n0, ebase, p = args
            kl = k - ws
            i = sload(widx_v, kl)
            slot = sload(wslt_v, kl)
            e = i - ebase
            m = (k - n0) & (CAP - 1)
            ev = jnp.full((L,), e, jnp.int32)
            for c in range(4):
                if from_tail:
                    vals = plsc.load_gather(tail_v, [acoef[c], rcoef[c], ev])
                else:
                    pv = jnp.full((L,), p, jnp.int32)
                    vals = plsc.load_gather(slab, [pv, acoef[c], rcoef[c], ev])
                stage_v[p, m, pl.ds(c * L, L)] = vals
            base16 = (m >> 4) * L
            cur = sl_v[p, pl.ds(base16, L)]
            sl_v[p, pl.ds(base16, L)] = jnp.where(
                lane == (m & (L - 1)), slot, cur)
            return args

        return serve

    serve_main = make_serve(False)
    serve_tail = make_serve(True)

    def sweep(tbl, si_hbm, ss_hbm, o_hbm):
        pltpu.sync_copy(o_hbm.at[pl.ds(0, OFFPAD)], off_v)
        ws = pl.multiple_of(sload(off_v, c0) & ~7, 8)
        pltpu.sync_copy(si_hbm.at[pl.ds(ws, WIN)], widx_v)
        pltpu.sync_copy(ss_hbm.at[pl.ds(ws, WIN)], wslt_v)
        for p in range(2):
            init_slots(p)
            pltpu.async_copy(stage_v.at[p], scr_hbm.at[sl_v.at[p]], sem_s)
        fire_load(tbl, c0)

        def chunk(cc, carry):
            p = cc & 1
            wait_load(tbl, cc)

            @pl.when(cc + 1 < c1m)
            def _():
                fire_load(tbl, cc + 1)

            pltpu.make_async_copy(
                stage_v.at[p], scr_hbm.at[sl_v.at[p]], sem_s).wait()
            init_slots(p)
            n0 = sload(off_v, cc)
            n1 = sload(off_v, cc + 1)
            lax.fori_loop(n0, n1, serve_main, (ws, n0, cc * CE, p))
            pltpu.async_copy(stage_v.at[p], scr_hbm.at[sl_v.at[p]], sem_s)
            return carry

        lax.fori_loop(c0, c1m, chunk, 0)
        for p in range(2):
            pltpu.make_async_copy(
                stage_v.at[p], scr_hbm.at[sl_v.at[p]], sem_s).wait()

        @pl.when(c1 == NCHK)
        def _():
            for a in range(8):
                pltpu.async_copy(
                    tbl.at[pl.ds(a * 8, 8), pl.ds(NFULL * CE, TAILW)],
                    tail_v.at[a], sem_l)
            for a in range(8):
                pltpu.make_async_copy(
                    tbl.at[pl.ds(a * 8, 8), pl.ds(NFULL * CE, TAILW)],
                    tail_v.at[a], sem_l).wait()
            init_slots(0)
            n0 = sload(off_v, NFULL)
            n1 = sload(off_v, NFULL + 1)
            lax.fori_loop(n0, n1, serve_tail,
                          (ws, n0, NFULL * CE, 0))
            pltpu.sync_copy(stage_v.at[0], scr_hbm.at[sl_v.at[0]])

    sweep(ent_hbm, sie_hbm, sse_hbm, offe_hbm)
    sweep(rel_hbm, sir_hbm, ssr_hbm, offr_hbm)


def _score_body(scr_hbm, out_hbm, hbuf, rbuf, tbuf, score_v, tpose_v,
                sem_h, sem_r, sem_t):
    wid = lax.axis_index("s") * NC + lax.axis_index("c")
    lane = lax.iota(jnp.int32, L)
    for j in range(BPW // CH):
        s0 = wid * BPW + j * CH
        cph = pltpu.async_copy(scr_hbm.at[pl.ds(s0, CH)], hbuf, sem_h)
        cpt = pltpu.async_copy(scr_hbm.at[pl.ds(B + s0, CH)], tbuf, sem_t)
        cpr = pltpu.async_copy(scr_hbm.at[pl.ds(2 * B + s0, CH)], rbuf, sem_r)
        cph.wait()
        cpt.wait()
        cpr.wait()

        def compute(g, carry):
            for ii in range(L):
                i = g * L + ii
                acc = None
                for c in range(D // L):
                    h = hbuf[i, pl.ds(c * L, L)]
                    r = rbuf[i, pl.ds(c * L, L)]
                    t = tbuf[i, pl.ds(c * L, L)]
                    a = jnp.abs(h + r - t)
                    acc = a if acc is None else acc + a
                tpose_v[ii, pl.ds(0, L)] = acc
            svec = None
            for c in range(L):
                part = plsc.load_gather(
                    tpose_v, [lane, jnp.full((L,), c, jnp.int32)])
                svec = part if svec is None else svec + part
            score_v[pl.ds(g * L, L)] = svec
            return carry

        lax.fori_loop(0, CH // L, compute, 0)
        pltpu.sync_copy(score_v, out_hbm.at[pl.ds(s0, CH)])


def kernel(sample, entity_embedding, relation_embedding):
    ent_t = entity_embedding.T
    rel_t = relation_embedding.T
    idx_e = jnp.concatenate([sample[:, 0], sample[:, 2]])
    perm_e = jnp.argsort(idx_e)
    sie = jnp.concatenate([idx_e[perm_e],
                           jnp.full((WIN,), jnp.int32(2**30))])
    sse = jnp.concatenate([perm_e.astype(jnp.int32),
                           jnp.zeros((WIN,), jnp.int32)])
    idx_r = sample[:, 1]
    perm_r = jnp.argsort(idx_r)
    sir = jnp.concatenate([idx_r[perm_r],
                           jnp.full((WIN,), jnp.int32(2**30))])
    ssr = jnp.concatenate([(perm_r + 2 * B).astype(jnp.int32),
                           jnp.zeros((WIN,), jnp.int32)])
    cnt_e = jnp.zeros((NCHK,), jnp.int32).at[idx_e // CE].add(1)
    cnt_r = jnp.zeros((NCHK,), jnp.int32).at[idx_r // CE].add(1)
    offe = jnp.concatenate([
        jnp.zeros((1,), jnp.int32), jnp.cumsum(cnt_e, dtype=jnp.int32),
        jnp.full((OFFPAD - NCHK - 1,), jnp.int32(2 * B))])
    offr = jnp.concatenate([
        jnp.zeros((1,), jnp.int32), jnp.cumsum(cnt_r, dtype=jnp.int32),
        jnp.full((OFFPAD - NCHK - 1,), jnp.int32(B))])

    mesh = plsc.VectorSubcoreMesh(
        core_axis_name="c", subcore_axis_name="s",
        num_cores=NC, num_subcores=NS)
    params = pltpu.CompilerParams(
        needs_layout_passes=False, use_tc_tiling_on_sc=True)

    gathered = pl.kernel(
        _sweep_body,
        out_type=jax.ShapeDtypeStruct((SCR, 2 * D), jnp.float32),
        mesh=mesh,
        compiler_params=params,
        scratch_types=[
            pltpu.VMEM((OFFPAD,), jnp.int32),
            pltpu.VMEM((WIN,), jnp.int32),
            pltpu.VMEM((WIN,), jnp.int32),
            pltpu.VMEM((2, 8, 8, CE), jnp.float32),
            pltpu.VMEM((8, 8, TAILW), jnp.float32),
            pltpu.VMEM((2, CAP, 2 * D), jnp.float32),
            pltpu.VMEM((2, CAP), jnp.int32),
            pltpu.SemaphoreType.DMA,
            pltpu.SemaphoreType.DMA,
        ],
    )(ent_t, rel_t, sie, sse, offe, sir, ssr, offr)

    score = pl.kernel(
        _score_body,
        out_type=jax.ShapeDtypeStruct((B,), jnp.float32),
        mesh=mesh,
        compiler_params=pltpu.CompilerParams(
            needs_layout_passes=False, use_tc_tiling_on_sc=False),
        scratch_types=[
            pltpu.VMEM((CH, 2 * D), jnp.float32),
            pltpu.VMEM((CH, 2 * D), jnp.float32),
            pltpu.VMEM((CH, 2 * D), jnp.float32),
            pltpu.VMEM((CH,), jnp.float32),
            pltpu.VMEM((L, L + 1), jnp.float32),
            pltpu.SemaphoreType.DMA,
            pltpu.SemaphoreType.DMA,
            pltpu.SemaphoreType.DMA,
        ],
    )(gathered)
    return score.reshape(B, 1)
